# baseline (speedup 1.0000x reference)
import functools

import jax
import jax.numpy as jnp
from jax import lax
from jax.experimental import pallas as pl
from jax.experimental.pallas import tpu as pltpu
from jax.experimental.pallas import tpu_sc as plsc

B = 16384
D = 64
NC = 2
NS = 16
NW = NC * NS
BPW = B // NW
L = 16
CHUNKS = BPW // L


def _svdpp_body(uid_hbm, iid_hbm, uemb_hbm, iemb_hbm, ubias_hbm, ibias_hbm,
                gbias_hbm, out_hbm,
                idx_u, idx_i, u_rows, v_rows, ub_v, ib_v, gb_v, out_v,
                sem_u, sem_i, sem_ub, sem_ib):
    wid = lax.axis_index("s") * NC + lax.axis_index("c")
    base = wid * BPW

    pltpu.sync_copy(uid_hbm.at[pl.ds(base, BPW)], idx_u)
    pltpu.sync_copy(iid_hbm.at[pl.ds(base, BPW)], idx_i)

    cu = pltpu.async_copy(uemb_hbm.at[idx_u], u_rows, sem_u)
    ci = pltpu.async_copy(iemb_hbm.at[idx_i], v_rows, sem_i)
    cub = pltpu.async_copy(ubias_hbm.at[idx_u], ub_v, sem_ub)
    cib = pltpu.async_copy(ibias_hbm.at[idx_i], ib_v, sem_ib)
    pltpu.sync_copy(gbias_hbm, gb_v.at[pl.ds(0, 1)])
    cu.wait()
    ci.wait()
    cub.wait()
    cib.wait()

    lane = lax.iota(jnp.int32, L)
    gb = gb_v[pl.ds(0, L)][0]

    def chunk(c, carry):
        b0 = c * L
        rows = b0 + lane
        acc = jnp.zeros((L,), jnp.float32)
        for d in range(D):
            dcol = jnp.full((L,), d, jnp.int32)
            acc = acc + (plsc.load_gather(u_rows, [rows, dcol])
                         * plsc.load_gather(v_rows, [rows, dcol]))
        acc = acc + ub_v[pl.ds(b0, L)] + ib_v[pl.ds(b0, L)] + gb
        out_v[pl.ds(b0, L)] = acc
        return carry

    lax.fori_loop(0, CHUNKS, chunk, 0)
    pltpu.sync_copy(out_v, out_hbm.at[pl.ds(base, BPW)])


_svdpp = functools.partial(
    pl.kernel,
    mesh=plsc.VectorSubcoreMesh(core_axis_name="c", subcore_axis_name="s"),
    out_type=jax.ShapeDtypeStruct((B,), jnp.float32),
    compiler_params=pltpu.CompilerParams(needs_layout_passes=False,
                                         use_tc_tiling_on_sc=False),
    scratch_types=[
        pltpu.VMEM((BPW,), jnp.int32),
        pltpu.VMEM((BPW,), jnp.int32),
        pltpu.VMEM((BPW, D), jnp.float32),
        pltpu.VMEM((BPW, D), jnp.float32),
        pltpu.VMEM((BPW,), jnp.float32),
        pltpu.VMEM((BPW,), jnp.float32),
        pltpu.VMEM((L,), jnp.float32),
        pltpu.VMEM((BPW,), jnp.float32),
        pltpu.SemaphoreType.DMA,
        pltpu.SemaphoreType.DMA,
        pltpu.SemaphoreType.DMA,
        pltpu.SemaphoreType.DMA,
    ],
)(_svdpp_body)


def kernel(user_ids, item_ids, user_embedding, item_embedding, user_bias,
           item_bias, global_bias):
    uid = user_ids.astype(jnp.int32)
    iid = item_ids.astype(jnp.int32)
    ub = user_bias[:, 0]
    ib = item_bias[:, 0]
    return _svdpp(uid, iid, user_embedding, item_embedding, ub, ib,
                  global_bias)


# speedup vs baseline: 1.0027x; 1.0027x over previous
import functools

import jax
import jax.numpy as jnp
from jax import lax
from jax.experimental import pallas as pl
from jax.experimental.pallas import tpu as pltpu
from jax.experimental.pallas import tpu_sc as plsc

B = 16384
D = 64
NC = 2
NS = 16
NW = NC * NS
BPW = B // NW
L = 16

_MESH = plsc.VectorSubcoreMesh(core_axis_name="c", subcore_axis_name="s")
_PARAMS = pltpu.CompilerParams(needs_layout_passes=False,
                               use_tc_tiling_on_sc=False)


def _user_body(uid_hbm, iid_hbm, uemb_hbm, ubias_hbm, ibias_hbm, gbias_hbm,
               bsum_hbm, urows_hbm,
               idx_u, idx_i, u_rows, ub_v, ib_v, gb_v, bsum_v,
               sem_u, sem_ub, sem_ib):
    wid = lax.axis_index("s") * NC + lax.axis_index("c")
    base = wid * BPW

    pltpu.sync_copy(uid_hbm.at[pl.ds(base, BPW)], idx_u)
    pltpu.sync_copy(iid_hbm.at[pl.ds(base, BPW)], idx_i)

    cu = pltpu.async_copy(uemb_hbm.at[idx_u], u_rows, sem_u)
    cub = pltpu.async_copy(ubias_hbm.at[idx_u], ub_v, sem_ub)
    cib = pltpu.async_copy(ibias_hbm.at[idx_i], ib_v, sem_ib)
    pltpu.sync_copy(gbias_hbm, gb_v.at[pl.ds(0, 1)])
    cub.wait()
    cib.wait()

    gb = gb_v[pl.ds(0, L)][0]

    def bias_sum(k, carry):
        sl = pl.ds(k * L, L)
        bsum_v[sl] = ub_v[sl] + ib_v[sl] + gb
        return carry

    lax.fori_loop(0, BPW // L, bias_sum, 0)
    pltpu.sync_copy(bsum_v, bsum_hbm.at[pl.ds(base, BPW)])
    cu.wait()
    pltpu.sync_copy(u_rows, urows_hbm.at[pl.ds(base, BPW)])


def _item_body(iid_hbm, iemb_hbm, urows_hbm, bsum_hbm, out_hbm,
               idx_i, u_rows, v_rows, bsum_v, out_v,
               sem_i, sem_u, sem_b):
    wid = lax.axis_index("s") * NC + lax.axis_index("c")
    base = wid * BPW

    pltpu.sync_copy(iid_hbm.at[pl.ds(base, BPW)], idx_i)
    ci = pltpu.async_copy(iemb_hbm.at[idx_i], v_rows, sem_i)
    cu = pltpu.async_copy(urows_hbm.at[pl.ds(base, BPW)], u_rows, sem_u)
    cb = pltpu.async_copy(bsum_hbm.at[pl.ds(base, BPW)], bsum_v, sem_b)
    ci.wait()
    cu.wait()
    cb.wait()

    lane = lax.iota(jnp.int32, L)

    def chunk(c, carry):
        b0 = c * L
        rows = b0 + lane
        acc = jnp.zeros((L,), jnp.float32)
        for d in range(D):
            dcol = jnp.full((L,), d, jnp.int32)
            acc = acc + (plsc.load_gather(u_rows, [rows, dcol])
                         * plsc.load_gather(v_rows, [rows, dcol]))
        sl = pl.ds(b0, L)
        out_v[sl] = acc + bsum_v[sl]
        return carry

    lax.fori_loop(0, BPW // L, chunk, 0)
    pltpu.sync_copy(out_v, out_hbm.at[pl.ds(base, BPW)])


_user_k = functools.partial(
    pl.kernel,
    mesh=_MESH,
    out_type=(jax.ShapeDtypeStruct((B,), jnp.float32),
              jax.ShapeDtypeStruct((B, D), jnp.float32)),
    compiler_params=_PARAMS,
    scratch_types=[
        pltpu.VMEM((BPW,), jnp.int32),
        pltpu.VMEM((BPW,), jnp.int32),
        pltpu.VMEM((BPW, D), jnp.float32),
        pltpu.VMEM((BPW,), jnp.float32),
        pltpu.VMEM((BPW,), jnp.float32),
        pltpu.VMEM((L,), jnp.float32),
        pltpu.VMEM((BPW,), jnp.float32),
        pltpu.SemaphoreType.DMA,
        pltpu.SemaphoreType.DMA,
        pltpu.SemaphoreType.DMA,
    ],
)(_user_body)


_item_k = functools.partial(
    pl.kernel,
    mesh=_MESH,
    out_type=jax.ShapeDtypeStruct((B,), jnp.float32),
    compiler_params=_PARAMS,
    scratch_types=[
        pltpu.VMEM((BPW,), jnp.int32),
        pltpu.VMEM((BPW, D), jnp.float32),
        pltpu.VMEM((BPW, D), jnp.float32),
        pltpu.VMEM((BPW,), jnp.float32),
        pltpu.VMEM((BPW,), jnp.float32),
        pltpu.SemaphoreType.DMA,
        pltpu.SemaphoreType.DMA,
        pltpu.SemaphoreType.DMA,
    ],
)(_item_body)


def kernel(user_ids, item_ids, user_embedding, item_embedding, user_bias,
           item_bias, global_bias):
    uid = user_ids.astype(jnp.int32)
    iid = item_ids.astype(jnp.int32)
    ub = user_bias[:, 0]
    ib = item_bias[:, 0]
    bsum, urows = _user_k(uid, iid, user_embedding, ub, ib, global_bias)
    return _item_k(iid, item_embedding, urows, bsum)


# speedup vs baseline: 1.4884x; 1.4845x over previous
import functools

import jax
import jax.numpy as jnp
from jax import lax
from jax.experimental import pallas as pl
from jax.experimental.pallas import tpu as pltpu
from jax.experimental.pallas import tpu_sc as plsc

B = 16384
D = 64
V = 1000000
NC = 2
NS = 16
NW = NC * NS
BPW = B // NW
L = 16
CW = 512
NCH = 61
RANGE = NCH * CW
TAIL_LO = 1952 * CW
MAIN_HI = 1953 * CW
SCAT_N = 128
DUMP = B
NG_IDS = B // L


def _bias_body(uid_hbm, iid_hbm, ubias_hbm, ibias_hbm, gbias_hbm, bsum_hbm,
               idx_u, idx_i, ub_v, ib_v, gb_v, bsum_v, sem_ub, sem_ib):
    wid = lax.axis_index("s") * NC + lax.axis_index("c")
    base = wid * BPW
    pltpu.sync_copy(uid_hbm.at[pl.ds(base, BPW)], idx_u)
    pltpu.sync_copy(iid_hbm.at[pl.ds(base, BPW)], idx_i)
    cub = pltpu.async_copy(ubias_hbm.at[idx_u], ub_v, sem_ub)
    cib = pltpu.async_copy(ibias_hbm.at[idx_i], ib_v, sem_ib)
    pltpu.sync_copy(gbias_hbm, gb_v.at[pl.ds(0, 1)])
    cub.wait()
    cib.wait()
    gb = gb_v[pl.ds(0, L)][0]

    def step(k, carry):
        sl = pl.ds(k * L, L)
        bsum_v[sl] = ub_v[sl] + ib_v[sl] + gb
        return carry

    lax.fori_loop(0, BPW // L, step, 0)
    pltpu.sync_copy(bsum_v, bsum_hbm.at[pl.ds(base, BPW)])


def _extract_body(uid_hbm, iid_hbm, ue_t_hbm, ie_t_hbm, ue_tail_hbm,
                  ie_tail_hbm, urows_hbm, vrows_hbm,
                  ids_v, mrow, mpos, block, sbuf, stage_c, stage_p, scat_idx,
                  slot_s, sem_blk, sem_sc):
    wid = lax.axis_index("s") * NC + lax.axis_index("c")
    lane = lax.iota(jnp.int32, L)
    last = wid == NW - 1
    lo = wid * RANGE
    nch = lax.select(last, jnp.int32(NCH + 1), jnp.int32(NCH))
    hi_match = lax.select(last, jnp.int32(V), lo + nch * CW)

    def reset_scat():
        for q in range(SCAT_N // L):
            scat_idx[pl.ds(q * L, L)] = jnp.full((L,), DUMP, jnp.int32)

    def flush(out_hbm):
        pltpu.async_copy(sbuf, out_hbm.at[scat_idx], sem_sc).wait()
        reset_scat()
        slot_s[0] = 0

    def extract_groups(ng, lo_c, hi_c, out_hbm):
        def gbody(g, carry):
            sl = pl.ds(g * L, L)
            rows = mrow[sl]
            pos = mpos[sl]
            m = jnp.logical_and(rows >= lo_c, rows < hi_c)
            cnt = plsc.all_reduce_population_count(m)[0]
            plsc.store_compressed(stage_c.at[:], rows - lo_c, mask=m)
            plsc.store_compressed(stage_p.at[:], pos, mask=m)

            def mbody(k, carry2):
                kk = jnp.zeros((L,), jnp.int32) + k
                col = plsc.load_gather(stage_c.at[:], [kk])
                bp = plsc.load_gather(stage_p.at[:], [kk])
                s = slot_s[0]
                for f in range(D // L):
                    vf = plsc.load_gather(block.at[:, :], [f * L + lane, col])
                    sbuf[s, pl.ds(f * L, L)] = vf
                plsc.store_scatter(scat_idx.at[:], [jnp.zeros((L,), jnp.int32) + s],
                                   bp, mask=lane == 0)

                @pl.when(s + 1 == SCAT_N)
                def _f():
                    flush(out_hbm)

                @pl.when(s + 1 < SCAT_N)
                def _g():
                    slot_s[0] = s + 1

                return carry2

            lax.fori_loop(0, cnt, mbody, 0)
            return carry

        lax.fori_loop(0, ng, gbody, 0)

    def process_table(ids_hbm, tab_hbm, tail_hbm, out_hbm):
        pltpu.sync_copy(ids_hbm, ids_v)
        reset_scat()
        slot_s[0] = 0

        def scan(g, cnt):
            rows = ids_v[pl.ds(g * L, L)]
            m = jnp.logical_and(rows >= lo, rows < hi_match)
            c = plsc.all_reduce_population_count(m)[0]
            plsc.store_compressed(mrow.at[pl.ds(cnt, L)], rows, mask=m)
            plsc.store_compressed(mpos.at[pl.ds(cnt, L)], g * L + lane, mask=m)
            return cnt + c

        cnt = lax.fori_loop(0, NG_IDS, scan, jnp.int32(0))
        mrow[pl.ds(cnt, L)] = jnp.full((L,), V, jnp.int32)
        mpos[pl.ds(cnt, L)] = jnp.full((L,), DUMP, jnp.int32)
        ng = (cnt + (L - 1)) // L

        def chunk_body(c, carry):
            c0 = pl.multiple_of(lo + c * CW, CW)
            pltpu.async_copy(tab_hbm.at[:, pl.ds(c0, CW)], block,
                             sem_blk).wait()
            extract_groups(ng, c0, c0 + CW, out_hbm)
            return carry

        lax.fori_loop(0, nch, chunk_body, 0)

        pltpu.async_copy(tail_hbm, block.at[:, pl.ds(0, 128)],
                         sem_blk).wait()
        tail_lo = lax.select(last, jnp.int32(MAIN_HI), jnp.int32(V))
        extract_groups(ng, tail_lo, jnp.int32(V), out_hbm)

        flush(out_hbm)

    process_table(uid_hbm, ue_t_hbm, ue_tail_hbm, urows_hbm)
    process_table(iid_hbm, ie_t_hbm, ie_tail_hbm, vrows_hbm)


def _dot_body(urows_hbm, vrows_hbm, bsum_hbm, out_hbm,
              u_v, v_v, b_v, o_v, sem_u, sem_v):
    wid = lax.axis_index("s") * NC + lax.axis_index("c")
    base = wid * BPW
    pltpu.sync_copy(bsum_hbm.at[pl.ds(base, BPW)], b_v)
    lane = lax.iota(jnp.int32, L)
    half = BPW // 2
    for h in range(2):
        hbase = base + h * half
        cu = pltpu.async_copy(urows_hbm.at[pl.ds(hbase, half)], u_v, sem_u)
        cv = pltpu.async_copy(vrows_hbm.at[pl.ds(hbase, half)], v_v, sem_v)
        cu.wait()
        cv.wait()

        def chunk(c, carry):
            b0 = c * L
            rows = b0 + lane
            acc = jnp.zeros((L,), jnp.float32)
            for d in range(D):
                dcol = jnp.full((L,), d, jnp.int32)
                acc = acc + (plsc.load_gather(u_v, [rows, dcol])
                             * plsc.load_gather(v_v, [rows, dcol]))
            sl = pl.ds(h * half + b0, L)
            o_v[sl] = acc + b_v[sl]
            return carry

        lax.fori_loop(0, half // L, chunk, 0)
    pltpu.sync_copy(o_v, out_hbm.at[pl.ds(base, BPW)])


_MESH = plsc.VectorSubcoreMesh(core_axis_name="c", subcore_axis_name="s")

_bias_k = functools.partial(
    pl.kernel,
    mesh=_MESH,
    out_type=jax.ShapeDtypeStruct((B,), jnp.float32),
    compiler_params=pltpu.CompilerParams(needs_layout_passes=False,
                                         use_tc_tiling_on_sc=False),
    scratch_types=[
        pltpu.VMEM((BPW,), jnp.int32),
        pltpu.VMEM((BPW,), jnp.int32),
        pltpu.VMEM((BPW,), jnp.float32),
        pltpu.VMEM((BPW,), jnp.float32),
        pltpu.VMEM((L,), jnp.float32),
        pltpu.VMEM((BPW,), jnp.float32),
        pltpu.SemaphoreType.DMA,
        pltpu.SemaphoreType.DMA,
    ],
)(_bias_body)

_extract_k = functools.partial(
    pl.kernel,
    mesh=_MESH,
    out_type=(jax.ShapeDtypeStruct((B + 8, 128), jnp.float32),
              jax.ShapeDtypeStruct((B + 8, 128), jnp.float32)),
    compiler_params=pltpu.CompilerParams(needs_layout_passes=False),
    scratch_types=[
        pltpu.VMEM((B,), jnp.int32),
        pltpu.VMEM((B + L,), jnp.int32),
        pltpu.VMEM((B + L,), jnp.int32),
        pltpu.VMEM((D, CW), jnp.float32),
        pltpu.VMEM((SCAT_N, 128), jnp.float32),
        pltpu.VMEM((L,), jnp.int32),
        pltpu.VMEM((L,), jnp.int32),
        pltpu.VMEM((SCAT_N,), jnp.int32),
        pltpu.SMEM((1,), jnp.int32),
        pltpu.SemaphoreType.DMA,
        pltpu.SemaphoreType.DMA,
    ],
)(_extract_body)

_dot_k = functools.partial(
    pl.kernel,
    mesh=_MESH,
    out_type=jax.ShapeDtypeStruct((B,), jnp.float32),
    compiler_params=pltpu.CompilerParams(needs_layout_passes=False),
    scratch_types=[
        pltpu.VMEM((BPW // 2, 128), jnp.float32),
        pltpu.VMEM((BPW // 2, 128), jnp.float32),
        pltpu.VMEM((BPW,), jnp.float32),
        pltpu.VMEM((BPW,), jnp.float32),
        pltpu.SemaphoreType.DMA,
        pltpu.SemaphoreType.DMA,
    ],
)(_dot_body)


def kernel(user_ids, item_ids, user_embedding, item_embedding, user_bias,
           item_bias, global_bias):
    uid = user_ids.astype(jnp.int32)
    iid = item_ids.astype(jnp.int32)
    ub = user_bias.reshape(-1)
    ib = item_bias.reshape(-1)
    ue_t = user_embedding.T
    ie_t = item_embedding.T
    ue_tail = jnp.pad(user_embedding[MAIN_HI:].T, ((0, 0), (0, 64)))
    ie_tail = jnp.pad(item_embedding[MAIN_HI:].T, ((0, 0), (0, 64)))
    bsum = _bias_k(uid, iid, ub, ib, global_bias)
    urows, vrows = _extract_k(uid, iid, ue_t, ie_t, ue_tail, ie_tail)
    return _dot_k(urows, vrows, bsum)
